# Initial kernel scaffold; baseline (speedup 1.0000x reference)
import jax
import jax.numpy as jnp
from jax.experimental import pallas as pl


def kernel(x, adjs, W2_1, W2_2, W3_1, W3_2, W4_1, W4_2, Wfc, bfc, Wqkv, bqkv, Wo, bo):
    raise NotImplementedError("write your pallas kernel here")


# speedup vs baseline: 5.9491x; 5.9491x over previous
import functools

import jax
import jax.numpy as jnp
from jax import lax
from jax.experimental import pallas as pl
from jax.experimental.pallas import tpu as pltpu
from jax.experimental.pallas import tpu_sc as plsc

_N = 10000
_E = 320000
_D = 128
_C = 64
_NH = 4
_DH = 16
_EPS = 1e-8

_NC = 2
_NS = 16
_NW = _NC * _NS
_EPW = _E // _NW
_K = 128
_JCH = 79
_EPWP = _JCH * _K
_NS_ROWS = 632
_NPAD = _NS_ROWS * _NS


def _prop_body(table_hbm, src_hbm, dst_hbm, zeros_hbm, zeros1_hbm, ones_hbm,
               out0, out1, deg0, deg1,
               src_c0, src_c1, dst_v, ones_v, rows0, rows1, acc, deg_acc,
               sem0, sem1, semi0, semi1, semd):
  c = lax.axis_index("c")
  s = lax.axis_index("s")
  wid = s * _NC + c
  r0 = s * _NS_ROWS

  pltpu.sync_copy(zeros_hbm, acc.at[pl.ds(r0, _NS_ROWS)])
  for m in range(5):
    @pl.when(s + 16 * m < _JCH)
    def _():
      pltpu.sync_copy(zeros1_hbm,
                      deg_acc.at[pl.ds((s + 16 * m) * _K, _K)])
  pltpu.sync_copy(dst_hbm.at[wid], dst_v)
  pltpu.sync_copy(ones_hbm, ones_v)
  plsc.subcore_barrier()

  ibase = wid * _EPWP

  def load_idx(buf, sem, j):
    pltpu.async_copy(src_hbm.at[pl.ds(ibase + j * _K, _K)], buf, sem)

  def wait_idx(buf, sem, j):
    pltpu.make_async_copy(src_hbm.at[pl.ds(ibase + j * _K, _K)], buf,
                          sem).wait()

  def gather(ibuf, rbuf, sem):
    pltpu.async_copy(table_hbm.at[ibuf], rbuf, sem)

  def wait_gather(ibuf, rbuf, sem):
    pltpu.make_async_copy(table_hbm.at[ibuf], rbuf, sem).wait()

  def scatter(rbuf, j):
    pltpu.sync_copy(rbuf, acc.at[dst_v.at[j]], add=True)
    pltpu.async_copy(ones_v, deg_acc.at[dst_v.at[j]], semd, add=True)

  pltpu.sync_copy(src_hbm.at[pl.ds(ibase, _K)], src_c0)
  gather(src_c0, rows0, sem0)
  load_idx(src_c1, semi1, 1)

  @pl.loop(0, _JCH - 4, step=2)
  def _(j):
    wait_idx(src_c1, semi1, j + 1)
    gather(src_c1, rows1, sem1)
    wait_gather(src_c0, rows0, sem0)
    load_idx(src_c0, semi0, j + 2)
    scatter(rows0, j)
    wait_idx(src_c0, semi0, j + 2)
    gather(src_c0, rows0, sem0)
    wait_gather(src_c1, rows1, sem1)
    load_idx(src_c1, semi1, j + 3)
    scatter(rows1, j + 1)

  wait_idx(src_c1, semi1, _JCH - 2)
  gather(src_c1, rows1, sem1)
  wait_gather(src_c0, rows0, sem0)
  load_idx(src_c0, semi0, _JCH - 1)
  scatter(rows0, _JCH - 3)
  wait_idx(src_c0, semi0, _JCH - 1)
  gather(src_c0, rows0, sem0)
  wait_gather(src_c1, rows1, sem1)
  scatter(rows1, _JCH - 2)
  wait_gather(src_c0, rows0, sem0)
  scatter(rows0, _JCH - 1)

  @pl.loop(0, _JCH)
  def _(j):
    pltpu.make_async_copy(ones_v, deg_acc.at[dst_v.at[0]], semd).wait()

  plsc.subcore_barrier()

  @pl.when(c == 0)
  def _():
    pltpu.sync_copy(acc.at[pl.ds(r0, _NS_ROWS)], out0.at[pl.ds(r0, _NS_ROWS)])
    for m in range(5):
      @pl.when(s + 16 * m < _JCH)
      def _():
        d0s = (s + 16 * m) * _K
        pltpu.sync_copy(deg_acc.at[pl.ds(d0s, _K)], deg0.at[pl.ds(d0s, _K)])

  @pl.when(c == 1)
  def _():
    pltpu.sync_copy(acc.at[pl.ds(r0, _NS_ROWS)], out1.at[pl.ds(r0, _NS_ROWS)])
    for m in range(5):
      @pl.when(s + 16 * m < _JCH)
      def _():
        d1s = (s + 16 * m) * _K
        pltpu.sync_copy(deg_acc.at[pl.ds(d1s, _K)], deg1.at[pl.ds(d1s, _K)])


@functools.lru_cache(maxsize=None)
def _get_prop_call():
  return pl.kernel(
      _prop_body,
      out_type=(jax.ShapeDtypeStruct((_NPAD, _D), jnp.float32),
                jax.ShapeDtypeStruct((_NPAD, _D), jnp.float32),
                jax.ShapeDtypeStruct((_NPAD,), jnp.float32),
                jax.ShapeDtypeStruct((_NPAD,), jnp.float32)),
      mesh=plsc.VectorSubcoreMesh(core_axis_name="c", subcore_axis_name="s",
                                  num_cores=_NC, num_subcores=_NS),
      scratch_types=(
          pltpu.VMEM((_K,), jnp.int32),
          pltpu.VMEM((_K,), jnp.int32),
          pltpu.VMEM((_JCH, _K), jnp.int32),
          pltpu.VMEM((_K,), jnp.float32),
          pltpu.VMEM((_K, _D), jnp.float32),
          pltpu.VMEM((_K, _D), jnp.float32),
          pltpu.VMEM_SHARED((_NPAD, _D), jnp.float32),
          pltpu.VMEM_SHARED((_NPAD,), jnp.float32),
          pltpu.SemaphoreType.DMA,
          pltpu.SemaphoreType.DMA,
          pltpu.SemaphoreType.DMA,
          pltpu.SemaphoreType.DMA,
          pltpu.SemaphoreType.DMA,
      ),
      name="sc_segment_sum",
  )


def _mm(a, b):
  return lax.dot_general(a, b, (((1,), (0,)), ((), ())),
                         preferred_element_type=jnp.float32)


def _minmax(h):
  mn = jnp.min(h, axis=1, keepdims=True)
  mx = jnp.max(h, axis=1, keepdims=True)
  return (h - mn) / (mx - mn + _EPS)


def _log_softmax(z):
  m = jnp.max(z, axis=1, keepdims=True)
  zm = z - m
  return zm - jnp.log(jnp.sum(jnp.exp(zm), axis=1, keepdims=True))


def _combine_prop(p0, p1, d0, d1):
  deg = jnp.maximum(d0 + d1, 1.0)
  return (p0 + p1) / deg


def _stage_a_body(x_ref, wfct_ref, bfc_ref, w31_ref, w32_ref, w41_ref,
                  w42_ref, xn_ref, xr_ref, out2_ref, out3_ref):
  xn = _minmax(x_ref[...])
  xn_ref[...] = xn
  xr = _mm(xn, wfct_ref[...]) + bfc_ref[...]
  xr_ref[...] = xr
  h2 = jnp.maximum(_minmax(_mm(xn, w31_ref[...])), 0.0)
  out2_ref[...] = _log_softmax(_mm(0.5 * h2 + 0.5 * xr, w32_ref[...]))
  h3 = jnp.maximum(_minmax(_mm(jnp.tanh(xn), w41_ref[...])), 0.0)
  out3_ref[...] = _log_softmax(_mm(jnp.tanh(0.5 * h3 + 0.5 * xr),
                                   w42_ref[...]))


def _stage_b_body(p0_ref, p1_ref, dg0_ref, dg1_ref, xn_ref, w21_ref, h_ref):
  pr = _combine_prop(p0_ref[...], p1_ref[...], dg0_ref[...], dg1_ref[...])
  xn = xn_ref[...]
  h_ref[...] = jnp.maximum(_minmax(_mm(0.5 * pr + 0.5 * xn, w21_ref[...])),
                           0.0)


def _stage_c_body(p0_ref, p1_ref, dg0_ref, dg1_ref, xr_ref, out2_ref,
                  out3_ref, w22_ref, wqt_ref, wkt_ref, wvt_ref, bq_ref,
                  bk_ref, bv_ref, wot_ref, bo_ref, out_ref):
  pr = _combine_prop(p0_ref[...], p1_ref[...], dg0_ref[...], dg1_ref[...])
  x1 = _log_softmax(_mm(0.5 * pr + 0.5 * xr_ref[...], w22_ref[...]))
  xs = (x1, out2_ref[...], out3_ref[...])

  ci = lax.broadcasted_iota(jnp.int32, (_C, _NH), 0) // _DH
  hi = lax.broadcasted_iota(jnp.int32, (_C, _NH), 1)
  S = (ci == hi).astype(jnp.float32)
  hi2 = lax.broadcasted_iota(jnp.int32, (_NH, _C), 0)
  ci2 = lax.broadcasted_iota(jnp.int32, (_NH, _C), 1) // _DH
  ST = (hi2 == ci2).astype(jnp.float32)

  q = [_mm(t, wqt_ref[...]) + bq_ref[...] for t in xs]
  k = [_mm(t, wkt_ref[...]) + bk_ref[...] for t in xs]
  v = [_mm(t, wvt_ref[...]) + bv_ref[...] for t in xs]

  acc = None
  for l1 in range(3):
    lg = [_mm(q[l1] * k[l2], S) * 0.25 for l2 in range(3)]
    m = jnp.maximum(jnp.maximum(lg[0], lg[1]), lg[2])
    e = [jnp.exp(t - m) for t in lg]
    den = e[0] + e[1] + e[2]
    o = None
    for l2 in range(3):
      contrib = _mm(e[l2] / den, ST) * v[l2]
      o = contrib if o is None else o + contrib
    y = _mm(o, wot_ref[...]) + bo_ref[...]
    acc = y if acc is None else acc + y

  out_ref[...] = _log_softmax(acc * (1.0 / 3.0))


def _full(shape):
  return pl.BlockSpec(shape, lambda i: (0,) * len(shape))


def _rows(block, width):
  return pl.BlockSpec((block, width), lambda i: (i, 0))


_BA = 632
_BC = 400

_stage_a = pl.pallas_call(
    _stage_a_body,
    grid=(_NPAD // _BA,),
    in_specs=[_rows(_BA, _D), _full((_D, _D)), _full((1, _D)),
              _full((_D, _D)), _full((_D, _C)), _full((_D, _D)),
              _full((_D, _C))],
    out_specs=[_rows(_BA, _D), _rows(_BA, _D), _rows(_BA, _C),
               _rows(_BA, _C)],
    out_shape=[jax.ShapeDtypeStruct((_NPAD, _D), jnp.float32),
               jax.ShapeDtypeStruct((_NPAD, _D), jnp.float32),
               jax.ShapeDtypeStruct((_NPAD, _C), jnp.float32),
               jax.ShapeDtypeStruct((_NPAD, _C), jnp.float32)],
)

_stage_b = pl.pallas_call(
    _stage_b_body,
    grid=(_NPAD // _BA,),
    in_specs=[_rows(_BA, _D), _rows(_BA, _D), _rows(_BA, 1), _rows(_BA, 1),
              _rows(_BA, _D), _full((_D, _D))],
    out_specs=_rows(_BA, _D),
    out_shape=jax.ShapeDtypeStruct((_NPAD, _D), jnp.float32),
)

_stage_c = pl.pallas_call(
    _stage_c_body,
    grid=(_N // _BC,),
    in_specs=[_rows(_BC, _D), _rows(_BC, _D), _rows(_BC, 1), _rows(_BC, 1),
              _rows(_BC, _D), _rows(_BC, _C), _rows(_BC, _C),
              _full((_D, _C)), _full((_C, _C)), _full((_C, _C)),
              _full((_C, _C)), _full((1, _C)), _full((1, _C)),
              _full((1, _C)), _full((_C, _C)), _full((1, _C))],
    out_specs=_rows(_BC, _C),
    out_shape=jax.ShapeDtypeStruct((_N, _C), jnp.float32),
)


def kernel(x, adjs, W2_1, W2_2, W3_1, W3_2, W4_1, W4_2, Wfc, bfc, Wqkv,
           bqkv, Wo, bo):
  x_pad = jnp.pad(x, ((0, _NPAD - _N), (0, 0)))
  pad_e = _EPWP - _EPW
  src1 = jnp.pad(adjs[0].reshape(_NW, _EPW),
                 ((0, 0), (0, pad_e))).reshape(_NW * _EPWP)
  dst3 = jnp.pad(adjs[1].reshape(_NW, _EPW), ((0, 0), (0, pad_e)),
                 constant_values=_NPAD - 1).reshape(_NW, _JCH, _K)
  zeros = jnp.zeros((_NS_ROWS, _D), jnp.float32)
  zeros1 = jnp.zeros((_K,), jnp.float32)
  ones1 = jnp.ones((_K,), jnp.float32)

  wfct = Wfc.T
  bfc2 = bfc.reshape(1, _D)
  wqt = Wqkv[:_C].T
  wkt = Wqkv[_C:2 * _C].T
  wvt = Wqkv[2 * _C:].T
  bq = bqkv[:_C].reshape(1, _C)
  bk = bqkv[_C:2 * _C].reshape(1, _C)
  bv = bqkv[2 * _C:].reshape(1, _C)
  wot = Wo.T
  bo2 = bo.reshape(1, _C)

  prop_call = _get_prop_call()
  xn, xr, out2, out3 = _stage_a(x_pad, wfct, bfc2, W3_1, W3_2, W4_1, W4_2)
  p0, p1, dga0, dga1 = prop_call(xn, src1, dst3, zeros, zeros1, ones1)
  dg0 = dga0.reshape(_NPAD, 1)
  dg1 = dga1.reshape(_NPAD, 1)
  h = _stage_b(p0, p1, dg0, dg1, xn, W2_1)
  q0, q1, dgb0, dgb1 = prop_call(h, src1, dst3, zeros, zeros1, ones1)
  return _stage_c(q0, q1, dgb0.reshape(_NPAD, 1), dgb1.reshape(_NPAD, 1),
                  xr, out2, out3, W2_2, wqt, wkt, wvt, bq, bk, bv, wot, bo2)
